# baseline (speedup 1.0000x reference)
import functools

import numpy as np
import jax
import jax.numpy as jnp
from jax import lax
from jax.experimental import pallas as pl
from jax.experimental.pallas import tpu as pltpu
from jax.experimental.pallas import tpu_sc as plsc

V, D, B, L, OUT = 100000, 300, 1024, 200, 256
NC, NS = 2, 16
NW = NC * NS
R = 2 * B
ROWS_W = R // NW
LP = 208
G = 104
NCHUNK = LP // G
COUNTS = (G, L - G)
NCHUNKS_W = ROWS_W * NCHUNK
NJ = OUT // 32
MBLK = 800

_PERM = np.empty((OUT,), np.int32)
for _j in range(NJ):
  _PERM[32 * _j + 0:32 * _j + 32:2] = np.arange(16) + 32 * _j
  _PERM[32 * _j + 1:32 * _j + 32:2] = np.arange(16) + 32 * _j + 16


NSTRM = LP // 16


def _pool_kernel(idx_hbm, tw_hbm, bias_hbm, out_hbm, idx_v, buf0, buf1,
                 bias_v, obuf_v, sem0, sem1):
  wid = lax.axis_index("s") * NC + lax.axis_index("c")
  idx_base = wid * (ROWS_W * LP)

  pltpu.sync_copy(idx_hbm.at[pl.ds(idx_base, ROWS_W * LP)], idx_v)
  pltpu.sync_copy(bias_hbm, bias_v)

  bufs = (buf0, buf1)
  sems = (sem0, sem1)

  def row_streams(b, slot, op):
    base = b * LP
    for i in range(NSTRM):
      idxv = idx_v[pl.ds(base + 16 * i, 16)]
      cp = pltpu.make_async_copy(
          tw_hbm.at[idxv], bufs[slot].at[pl.ds(16 * i, 16)], sems[slot])
      op(cp)

  row_streams(jnp.int32(0), 0, lambda cp: cp.start())
  row_streams(jnp.int32(1), 1, lambda cp: cp.start())

  inv_l = jnp.float32(1.0 / L)

  def pair_body(bp, carry):
    for half in range(2):
      b = 2 * bp + half
      row_streams(b, half, lambda cp: cp.wait())

      @pl.when(b + 2 < ROWS_W)
      def _():
        row_streams(b + 2, half, lambda cp: cp.start())

      buf = bufs[half]

      def accum_body(l, acc):
        acc = list(acc)
        for j in range(NJ):
          v = plsc.bitcast(buf[l, pl.ds(32 * j, 32)], jnp.int32)
          lo = plsc.bitcast(lax.shift_left(v, 16), jnp.float32)
          hi = plsc.bitcast(
              lax.bitwise_and(v, jnp.int32(-65536)), jnp.float32)
          acc[2 * j] = acc[2 * j] + lo
          acc[2 * j + 1] = acc[2 * j + 1] + hi
        return tuple(acc)

      acc = lax.fori_loop(
          0, L, accum_body,
          tuple([jnp.zeros((16,), jnp.float32)] * (2 * NJ)))

      for c in range(2 * NJ):
        obuf_v[b, pl.ds(c * 16, 16)] = (
            acc[c] * inv_l + bias_v[pl.ds(c * 16, 16)])
    return carry

  lax.fori_loop(0, ROWS_W // 2, pair_body, jnp.int32(0))
  pltpu.sync_copy(obuf_v, out_hbm.at[pl.ds(wid * ROWS_W, ROWS_W)])


def _mm_kernel(x_ref, wt_ref, o_ref):
  o_ref[...] = jnp.dot(x_ref[...], wt_ref[...],
                       preferred_element_type=jnp.float32).astype(jnp.bfloat16)


@jax.jit
def _fused(idx_flat, table, wt, bias):
  tablew = pl.pallas_call(
      _mm_kernel,
      grid=(V // MBLK,),
      in_specs=[
          pl.BlockSpec((MBLK, D), lambda i: (i, 0)),
          pl.BlockSpec((D, OUT), lambda i: (0, 0)),
      ],
      out_specs=pl.BlockSpec((MBLK, OUT), lambda i: (i, 0)),
      out_shape=jax.ShapeDtypeStruct((V, OUT), jnp.bfloat16),
  )(table, wt)

  mesh = plsc.VectorSubcoreMesh(core_axis_name="c", subcore_axis_name="s",
                                num_cores=NC, num_subcores=NS)
  return pl.kernel(
      _pool_kernel,
      out_type=jax.ShapeDtypeStruct((R, OUT), jnp.float32),
      mesh=mesh,
      compiler_params=pltpu.CompilerParams(use_tc_tiling_on_sc=False,
                                           needs_layout_passes=False),
      scratch_types=[
          pltpu.VMEM((ROWS_W * LP,), jnp.int32),
          pltpu.VMEM((LP, OUT), jnp.bfloat16),
          pltpu.VMEM((LP, OUT), jnp.bfloat16),
          pltpu.VMEM((OUT,), jnp.float32),
          pltpu.VMEM((ROWS_W, OUT), jnp.float32),
          pltpu.SemaphoreType.DMA,
          pltpu.SemaphoreType.DMA,
      ],
  )(idx_flat, tablew, bias)


def kernel(inputs_1, inputs_2, table, W, b):
  idx = jnp.concatenate([inputs_1, inputs_2], axis=0)
  idx_flat = jnp.pad(idx, ((0, 0), (0, LP - L))).reshape(-1)
  perm = jnp.asarray(_PERM)
  out = _fused(idx_flat, table, W.T[:, perm], b)
  return out[:B], out[B:]


# speedup vs baseline: 2.0497x; 2.0497x over previous
import functools

import numpy as np
import jax
import jax.numpy as jnp
from jax import lax
from jax.experimental import pallas as pl
from jax.experimental.pallas import tpu as pltpu
from jax.experimental.pallas import tpu_sc as plsc

V, D, B, L, OUT = 100000, 300, 1024, 200, 256
NC, NS = 2, 16
NW = NC * NS
R = 2 * B
ROWS_W = R // NW
LP = 208
G = 104
NCHUNK = LP // G
COUNTS = (G, L - G)
NCHUNKS_W = ROWS_W * NCHUNK
NJ = OUT // 32
MBLK = 800

_PERM = np.empty((OUT,), np.int32)
for _j in range(NJ):
  _PERM[32 * _j + 0:32 * _j + 32:2] = np.arange(16) + 32 * _j
  _PERM[32 * _j + 1:32 * _j + 32:2] = np.arange(16) + 32 * _j + 16


def _pool_kernel(idx_hbm, tw_hbm, bias_hbm, out_hbm, idx_v, buf0, buf1,
                 bias_v, obuf_v, sem0, sem1):
  wid = lax.axis_index("s") * NC + lax.axis_index("c")
  idx_base = wid * (ROWS_W * LP)

  pltpu.sync_copy(idx_hbm.at[pl.ds(idx_base, ROWS_W * LP)], idx_v)
  pltpu.sync_copy(bias_hbm, bias_v)

  bufs = (buf0, buf1)
  sems = (sem0, sem1)

  def start_gather(flat_chunk, parity):
    pltpu.make_async_copy(
        tw_hbm.at[idx_v.at[pl.ds(flat_chunk * G, G)]],
        bufs[parity], sems[parity]).start()

  start_gather(jnp.int32(0), 0)

  inv_l = jnp.float32(1.0 / L)

  def row_body(b, carry):
    acc = [jnp.zeros((16,), jnp.float32)] * (2 * NJ)
    for k in range(NCHUNK):
      flat = b * NCHUNK + k
      p = k % 2
      pltpu.make_async_copy(
          tw_hbm.at[idx_v.at[pl.ds(flat * G, G)]],
          bufs[p], sems[p]).wait()

      @pl.when(flat + 1 < NCHUNKS_W)
      def _():
        start_gather(flat + 1, (k + 1) % 2)

      buf = bufs[p]

      def accum_body(l, acc):
        acc = list(acc)
        for j in range(NJ):
          v = plsc.bitcast(buf[l, pl.ds(32 * j, 32)], jnp.int32)
          lo = plsc.bitcast(lax.shift_left(v, 16), jnp.float32)
          hi = plsc.bitcast(
              lax.bitwise_and(v, jnp.int32(-65536)), jnp.float32)
          acc[2 * j] = acc[2 * j] + lo
          acc[2 * j + 1] = acc[2 * j + 1] + hi
        return tuple(acc)

      acc = lax.fori_loop(0, COUNTS[k], accum_body, tuple(acc))

    for c in range(2 * NJ):
      obuf_v[b, pl.ds(c * 16, 16)] = (
          acc[c] * inv_l + bias_v[pl.ds(c * 16, 16)])
    return carry

  lax.fori_loop(0, ROWS_W, row_body, jnp.int32(0))
  pltpu.sync_copy(obuf_v, out_hbm.at[pl.ds(wid * ROWS_W, ROWS_W)])


def _mm_kernel(x_ref, wt_ref, o_ref):
  o_ref[...] = jnp.dot(x_ref[...], wt_ref[...],
                       preferred_element_type=jnp.float32).astype(jnp.bfloat16)


@jax.jit
def _fused(idx_flat, table, wt, bias):
  tablew = pl.pallas_call(
      _mm_kernel,
      grid=(V // MBLK,),
      in_specs=[
          pl.BlockSpec((MBLK, D), lambda i: (i, 0)),
          pl.BlockSpec((D, OUT), lambda i: (0, 0)),
      ],
      out_specs=pl.BlockSpec((MBLK, OUT), lambda i: (i, 0)),
      out_shape=jax.ShapeDtypeStruct((V, OUT), jnp.bfloat16),
  )(table, wt)

  mesh = plsc.VectorSubcoreMesh(core_axis_name="c", subcore_axis_name="s",
                                num_cores=NC, num_subcores=NS)
  return pl.kernel(
      _pool_kernel,
      out_type=jax.ShapeDtypeStruct((R, OUT), jnp.float32),
      mesh=mesh,
      compiler_params=pltpu.CompilerParams(use_tc_tiling_on_sc=False,
                                           needs_layout_passes=False),
      scratch_types=[
          pltpu.VMEM((ROWS_W * LP,), jnp.int32),
          pltpu.VMEM((G, OUT), jnp.bfloat16),
          pltpu.VMEM((G, OUT), jnp.bfloat16),
          pltpu.VMEM((OUT,), jnp.float32),
          pltpu.VMEM((ROWS_W, OUT), jnp.float32),
          pltpu.SemaphoreType.DMA,
          pltpu.SemaphoreType.DMA,
      ],
  )(idx_flat, tablew, bias)


def kernel(inputs_1, inputs_2, table, W, b):
  idx = jnp.concatenate([inputs_1, inputs_2], axis=0)
  pad = (jnp.arange(R * (LP - L), dtype=jnp.int32) % V).reshape(R, LP - L)
  idx_flat = jnp.concatenate([idx, pad], axis=1).reshape(-1)
  perm = jnp.asarray(_PERM)
  out = _fused(idx_flat, table, W.T[:, perm], b)
  return out[:B], out[B:]


# speedup vs baseline: 2.2383x; 1.0920x over previous
import functools

import numpy as np
import jax
import jax.numpy as jnp
from jax import lax
from jax.experimental import pallas as pl
from jax.experimental.pallas import tpu as pltpu
from jax.experimental.pallas import tpu_sc as plsc

V, D, B, L, OUT = 100000, 300, 1024, 200, 256
NC, NS = 2, 16
NW = NC * NS
R = 2 * B
ROWS_W = R // NW
LP = 208
G = 104
NCHUNK = LP // G
COUNTS = (G, L - G)
NCHUNKS_W = ROWS_W * NCHUNK
NJ = OUT // 32
MBLK = 2000

_PERM = np.empty((OUT,), np.int32)
for _j in range(NJ):
  _PERM[32 * _j + 0:32 * _j + 32:2] = np.arange(16) + 32 * _j
  _PERM[32 * _j + 1:32 * _j + 32:2] = np.arange(16) + 32 * _j + 16


def _pool_kernel(idx_hbm, tw_hbm, bias_hbm, out_hbm, idx_v, buf0, buf1,
                 bias_v, obuf_v, sem0, sem1):
  wid = lax.axis_index("s") * NC + lax.axis_index("c")
  idx_base = wid * (ROWS_W * LP)

  pltpu.sync_copy(idx_hbm.at[pl.ds(idx_base, ROWS_W * LP)], idx_v)
  pltpu.sync_copy(bias_hbm, bias_v)

  bufs = (buf0, buf1)
  sems = (sem0, sem1)

  def start_gather(flat_chunk, parity):
    pltpu.make_async_copy(
        tw_hbm.at[idx_v.at[pl.ds(flat_chunk * G, G)]],
        bufs[parity], sems[parity]).start()

  start_gather(jnp.int32(0), 0)

  inv_l = jnp.float32(1.0 / L)

  def row_body(b, carry):
    acc = [jnp.zeros((16,), jnp.float32)] * (2 * NJ)
    for k in range(NCHUNK):
      flat = b * NCHUNK + k
      p = k % 2
      pltpu.make_async_copy(
          tw_hbm.at[idx_v.at[pl.ds(flat * G, G)]],
          bufs[p], sems[p]).wait()

      @pl.when(flat + 1 < NCHUNKS_W)
      def _():
        start_gather(flat + 1, (k + 1) % 2)

      buf = bufs[p]

      def accum_body(l, acc):
        acc = list(acc)
        for j in range(NJ):
          v = plsc.bitcast(buf[l, pl.ds(32 * j, 32)], jnp.int32)
          lo = plsc.bitcast(lax.shift_left(v, 16), jnp.float32)
          hi = plsc.bitcast(
              lax.bitwise_and(v, jnp.int32(-65536)), jnp.float32)
          acc[2 * j] = acc[2 * j] + lo
          acc[2 * j + 1] = acc[2 * j + 1] + hi
        return tuple(acc)

      acc = lax.fori_loop(0, COUNTS[k], accum_body, tuple(acc))

    for c in range(2 * NJ):
      obuf_v[b, pl.ds(c * 16, 16)] = (
          acc[c] * inv_l + bias_v[pl.ds(c * 16, 16)])
    return carry

  lax.fori_loop(0, ROWS_W, row_body, jnp.int32(0))
  pltpu.sync_copy(obuf_v, out_hbm.at[pl.ds(wid * ROWS_W, ROWS_W)])


def _mm_kernel(x_ref, wt_ref, o_ref):
  o_ref[...] = jnp.dot(x_ref[...], wt_ref[...],
                       preferred_element_type=jnp.float32).astype(jnp.bfloat16)


@jax.jit
def _fused(inputs_1, inputs_2, table, W, b):
  idx = jnp.concatenate([inputs_1, inputs_2], axis=0)
  pad = (jnp.arange(R * (LP - L), dtype=jnp.int32) % V).reshape(R, LP - L)
  idx_flat = jnp.concatenate([idx, pad], axis=1).reshape(-1)
  wt = W.T[:, jnp.asarray(_PERM)]

  tablew = pl.pallas_call(
      _mm_kernel,
      grid=(V // MBLK,),
      in_specs=[
          pl.BlockSpec((MBLK, D), lambda i: (i, 0)),
          pl.BlockSpec((D, OUT), lambda i: (0, 0)),
      ],
      out_specs=pl.BlockSpec((MBLK, OUT), lambda i: (i, 0)),
      out_shape=jax.ShapeDtypeStruct((V, OUT), jnp.bfloat16),
  )(table, wt)

  mesh = plsc.VectorSubcoreMesh(core_axis_name="c", subcore_axis_name="s",
                                num_cores=NC, num_subcores=NS)
  out = pl.kernel(
      _pool_kernel,
      out_type=jax.ShapeDtypeStruct((R, OUT), jnp.float32),
      mesh=mesh,
      compiler_params=pltpu.CompilerParams(use_tc_tiling_on_sc=False,
                                           needs_layout_passes=False),
      scratch_types=[
          pltpu.VMEM((ROWS_W * LP,), jnp.int32),
          pltpu.VMEM((G, OUT), jnp.bfloat16),
          pltpu.VMEM((G, OUT), jnp.bfloat16),
          pltpu.VMEM((OUT,), jnp.float32),
          pltpu.VMEM((ROWS_W, OUT), jnp.float32),
          pltpu.SemaphoreType.DMA,
          pltpu.SemaphoreType.DMA,
      ],
  )(idx_flat, tablew, b)
  return out[:B], out[B:]


def kernel(inputs_1, inputs_2, table, W, b):
  return _fused(inputs_1, inputs_2, table, W, b)


# speedup vs baseline: 2.3090x; 1.0316x over previous
import functools

import numpy as np
import jax
import jax.numpy as jnp
from jax import lax
from jax.experimental import pallas as pl
from jax.experimental.pallas import tpu as pltpu
from jax.experimental.pallas import tpu_sc as plsc

V, D, B, L, OUT = 100000, 300, 1024, 200, 256
NC, NS = 2, 16
NW = NC * NS
R = 2 * B
ROWS_W = R // NW
LP = 208
G = 104
NCHUNK = LP // G
COUNTS = (G, L - G)
NCHUNKS_W = ROWS_W * NCHUNK
NJ = OUT // 32
MBLK = 4000

_PERM = np.empty((OUT,), np.int32)
for _j in range(NJ):
  _PERM[32 * _j + 0:32 * _j + 32:2] = np.arange(16) + 32 * _j
  _PERM[32 * _j + 1:32 * _j + 32:2] = np.arange(16) + 32 * _j + 16


def _pool_kernel(idx_hbm, tw_hbm, bias_hbm, out_hbm, idx_v, buf0, buf1,
                 bias_v, obuf_v, sem0, sem1):
  wid = lax.axis_index("s") * NC + lax.axis_index("c")
  idx_base = wid * (ROWS_W * LP)

  pltpu.sync_copy(idx_hbm.at[pl.ds(idx_base, ROWS_W * LP)], idx_v)
  pltpu.sync_copy(bias_hbm, bias_v)

  bufs = (buf0, buf1)
  sems = (sem0, sem1)

  def start_gather(flat_chunk, parity):
    pltpu.make_async_copy(
        tw_hbm.at[idx_v.at[pl.ds(flat_chunk * G, G)]],
        bufs[parity], sems[parity]).start()

  start_gather(jnp.int32(0), 0)

  inv_l = jnp.float32(1.0 / L)

  def row_body(b, carry):
    acc = [jnp.zeros((16,), jnp.float32)] * (2 * NJ)
    for k in range(NCHUNK):
      flat = b * NCHUNK + k
      p = k % 2
      pltpu.make_async_copy(
          tw_hbm.at[idx_v.at[pl.ds(flat * G, G)]],
          bufs[p], sems[p]).wait()

      @pl.when(flat + 1 < NCHUNKS_W)
      def _():
        start_gather(flat + 1, (k + 1) % 2)

      buf = bufs[p]

      def accum_body(l, acc):
        acc = list(acc)
        for j in range(NJ):
          v = plsc.bitcast(buf[l, pl.ds(32 * j, 32)], jnp.int32)
          lo = plsc.bitcast(lax.shift_left(v, 16), jnp.float32)
          hi = plsc.bitcast(
              lax.bitwise_and(v, jnp.int32(-65536)), jnp.float32)
          acc[2 * j] = acc[2 * j] + lo
          acc[2 * j + 1] = acc[2 * j + 1] + hi
        return tuple(acc)

      acc = lax.fori_loop(0, COUNTS[k], accum_body, tuple(acc))

    for c in range(2 * NJ):
      obuf_v[b, pl.ds(c * 16, 16)] = (
          acc[c] * inv_l + bias_v[pl.ds(c * 16, 16)])
    return carry

  lax.fori_loop(0, ROWS_W, row_body, jnp.int32(0))
  pltpu.sync_copy(obuf_v, out_hbm.at[pl.ds(wid * ROWS_W, ROWS_W)])


def _mm_kernel(x_ref, wt_ref, o_ref):
  o_ref[...] = jnp.dot(x_ref[...], wt_ref[...],
                       preferred_element_type=jnp.float32).astype(jnp.bfloat16)


@jax.jit
def _fused(inputs_1, inputs_2, table, W, b):
  idx = jnp.concatenate([inputs_1, inputs_2], axis=0)
  pad = (jnp.arange(R * (LP - L), dtype=jnp.int32) % V).reshape(R, LP - L)
  idx_flat = jnp.concatenate([idx, pad], axis=1).reshape(-1)
  wt = W.T[:, jnp.asarray(_PERM)]

  tablew = pl.pallas_call(
      _mm_kernel,
      grid=(V // MBLK,),
      in_specs=[
          pl.BlockSpec((MBLK, D), lambda i: (i, 0)),
          pl.BlockSpec((D, OUT), lambda i: (0, 0)),
      ],
      out_specs=pl.BlockSpec((MBLK, OUT), lambda i: (i, 0)),
      out_shape=jax.ShapeDtypeStruct((V, OUT), jnp.bfloat16),
  )(table, wt)

  mesh = plsc.VectorSubcoreMesh(core_axis_name="c", subcore_axis_name="s",
                                num_cores=NC, num_subcores=NS)
  out = pl.kernel(
      _pool_kernel,
      out_type=jax.ShapeDtypeStruct((R, OUT), jnp.float32),
      mesh=mesh,
      compiler_params=pltpu.CompilerParams(use_tc_tiling_on_sc=False,
                                           needs_layout_passes=False),
      scratch_types=[
          pltpu.VMEM((ROWS_W * LP,), jnp.int32),
          pltpu.VMEM((G, OUT), jnp.bfloat16),
          pltpu.VMEM((G, OUT), jnp.bfloat16),
          pltpu.VMEM((OUT,), jnp.float32),
          pltpu.VMEM((ROWS_W, OUT), jnp.float32),
          pltpu.SemaphoreType.DMA,
          pltpu.SemaphoreType.DMA,
      ],
  )(idx_flat, tablew, b)
  return out[:B], out[B:]


def kernel(inputs_1, inputs_2, table, W, b):
  return _fused(inputs_1, inputs_2, table, W, b)


# speedup vs baseline: 2.8288x; 1.2251x over previous
import functools

import numpy as np
import jax
import jax.numpy as jnp
from jax import lax
from jax.experimental import pallas as pl
from jax.experimental.pallas import tpu as pltpu
from jax.experimental.pallas import tpu_sc as plsc

V, D, B, L, OUT = 100000, 300, 1024, 200, 256
NC, NS = 2, 16
NW = NC * NS
R = 2 * B
ROWS_W = R // NW
LP = 208
G = 104
NCHUNK = LP // G
COUNTS = (G, L - G)
NCHUNKS_W = ROWS_W * NCHUNK
NJ = OUT // 32
MBLK = 4000

_PERM = np.empty((OUT,), np.int32)
for _p in range(128):
  _PERM[_p] = 32 * (_p // 16) + (_p % 16)
  _PERM[128 + _p] = 32 * (_p // 16) + 16 + (_p % 16)


def _pool_kernel(idx_hbm, tw_hbm, bias_hbm, out_hbm, idx_v, buf0, buf1,
                 bias_v, obuf_v, sem0, sem1):
  wid = lax.axis_index("s") * NC + lax.axis_index("c")
  idx_base = wid * (ROWS_W * LP)

  pltpu.sync_copy(idx_hbm.at[pl.ds(idx_base, ROWS_W * LP)], idx_v)
  pltpu.sync_copy(bias_hbm, bias_v)

  bufs = (buf0, buf1)
  sems = (sem0, sem1)

  def start_gather(flat_chunk, parity):
    pltpu.make_async_copy(
        tw_hbm.at[idx_v.at[pl.ds(flat_chunk * G, G)]],
        bufs[parity], sems[parity]).start()

  start_gather(jnp.int32(0), 0)

  inv_l = jnp.float32(1.0 / L)

  def row_body(b, carry):
    acc = [jnp.zeros((16,), jnp.float32)] * (2 * NJ)
    for k in range(NCHUNK):
      flat = b * NCHUNK + k
      p = k % 2
      pltpu.make_async_copy(
          tw_hbm.at[idx_v.at[pl.ds(flat * G, G)]],
          bufs[p], sems[p]).wait()

      @pl.when(flat + 1 < NCHUNKS_W)
      def _():
        start_gather(flat + 1, (k + 1) % 2)

      buf = bufs[p]

      def accum_body(l, acc):
        acc = list(acc)
        for j in range(NJ):
          v = buf[l, pl.ds(16 * j, 16)]
          lo = plsc.bitcast(lax.shift_left(v, 16), jnp.float32)
          hi = plsc.bitcast(
              lax.bitwise_and(v, jnp.int32(-65536)), jnp.float32)
          acc[2 * j] = acc[2 * j] + lo
          acc[2 * j + 1] = acc[2 * j + 1] + hi
        return tuple(acc)

      acc = lax.fori_loop(0, COUNTS[k], accum_body, tuple(acc))

    for c in range(2 * NJ):
      obuf_v[b, pl.ds(c * 16, 16)] = (
          acc[c] * inv_l + bias_v[pl.ds(c * 16, 16)])
    return carry

  lax.fori_loop(0, ROWS_W, row_body, jnp.int32(0))
  pltpu.sync_copy(obuf_v, out_hbm.at[pl.ds(wid * ROWS_W, ROWS_W)])


def _mm_kernel(x_ref, wt_ref, o_ref):
  r = jnp.dot(x_ref[...], wt_ref[...], preferred_element_type=jnp.float32)
  def bf16_bits(x):
    return lax.bitcast_convert_type(
        x.astype(jnp.bfloat16).astype(jnp.float32), jnp.int32)
  lo = lax.shift_right_logical(bf16_bits(r[:, :OUT // 2]), 16)
  hi = lax.bitwise_and(bf16_bits(r[:, OUT // 2:]), jnp.int32(-65536))
  o_ref[...] = lax.bitwise_or(hi, lo)


@jax.jit
def _fused(inputs_1, inputs_2, table, W, b):
  idx = jnp.concatenate([inputs_1, inputs_2], axis=0)
  pad = (jnp.arange(R * (LP - L), dtype=jnp.int32) % V).reshape(R, LP - L)
  idx_flat = jnp.concatenate([idx, pad], axis=1).reshape(-1)
  wt = W.T[:, jnp.asarray(_PERM)]

  tablew = pl.pallas_call(
      _mm_kernel,
      grid=(V // MBLK,),
      in_specs=[
          pl.BlockSpec((MBLK, D), lambda i: (i, 0)),
          pl.BlockSpec((D, OUT), lambda i: (0, 0)),
      ],
      out_specs=pl.BlockSpec((MBLK, OUT // 2), lambda i: (i, 0)),
      out_shape=jax.ShapeDtypeStruct((V, OUT // 2), jnp.int32),
  )(table, wt)

  mesh = plsc.VectorSubcoreMesh(core_axis_name="c", subcore_axis_name="s",
                                num_cores=NC, num_subcores=NS)
  out = pl.kernel(
      _pool_kernel,
      out_type=jax.ShapeDtypeStruct((R, OUT), jnp.float32),
      mesh=mesh,
      compiler_params=pltpu.CompilerParams(use_tc_tiling_on_sc=False,
                                           needs_layout_passes=False),
      scratch_types=[
          pltpu.VMEM((ROWS_W * LP,), jnp.int32),
          pltpu.VMEM((G, OUT // 2), jnp.int32),
          pltpu.VMEM((G, OUT // 2), jnp.int32),
          pltpu.VMEM((OUT,), jnp.float32),
          pltpu.VMEM((ROWS_W, OUT), jnp.float32),
          pltpu.SemaphoreType.DMA,
          pltpu.SemaphoreType.DMA,
      ],
  )(idx_flat, tablew, b)
  return out[:B], out[B:]


def kernel(inputs_1, inputs_2, table, W, b):
  return _fused(inputs_1, inputs_2, table, W, b)


# speedup vs baseline: 2.8291x; 1.0001x over previous
import functools

import numpy as np
import jax
import jax.numpy as jnp
from jax import lax
from jax.experimental import pallas as pl
from jax.experimental.pallas import tpu as pltpu
from jax.experimental.pallas import tpu_sc as plsc

V, D, B, L, OUT = 100000, 300, 1024, 200, 256
NC, NS = 2, 16
NW = NC * NS
R = 2 * B
ROWS_W = B // (NW // 2)
G = 104
COUNTS = (G, L - G)
NJ = OUT // 32
MBLK = 4000

_PERM = np.empty((OUT,), np.int32)
for _p in range(128):
  _PERM[_p] = 32 * (_p // 16) + (_p % 16)
  _PERM[128 + _p] = 32 * (_p // 16) + 16 + (_p % 16)


def _pool_kernel(idx1_hbm, idx2_hbm, tw_hbm, bias_hbm, out1_hbm, out2_hbm,
                 idx_v, buf0, buf1, bias_v, obuf_v, sem0, sem1):
  wid = lax.axis_index("s") * NC + lax.axis_index("c")
  first = wid < (NW // 2)
  lw = lax.select(first, wid, wid - NW // 2)
  idx_base = lw * (ROWS_W * L)

  @pl.when(first)
  def _():
    pltpu.sync_copy(idx1_hbm.at[pl.ds(idx_base, ROWS_W * L)], idx_v)

  @pl.when(jnp.logical_not(first))
  def _():
    pltpu.sync_copy(idx2_hbm.at[pl.ds(idx_base, ROWS_W * L)], idx_v)

  pltpu.sync_copy(bias_hbm, bias_v)

  bufs = (buf0, buf1)
  sems = (sem0, sem1)

  def start_gather(b, k):
    pltpu.make_async_copy(
        tw_hbm.at[idx_v.at[pl.ds(b * L + k * G, COUNTS[k])]],
        bufs[k], sems[k]).start()

  start_gather(jnp.int32(0), 0)

  inv_l = jnp.float32(1.0 / L)

  def row_body(b, carry):
    acc = [jnp.zeros((16,), jnp.float32)] * (2 * NJ)
    for k in range(2):
      pltpu.make_async_copy(
          tw_hbm.at[idx_v.at[pl.ds(b * L + k * G, COUNTS[k])]],
          bufs[k], sems[k]).wait()

      if k == 0:
        start_gather(b, 1)
      else:
        @pl.when(b + 1 < ROWS_W)
        def _():
          start_gather(b + 1, 0)

      buf = bufs[k]

      def accum_body(l, acc):
        acc = list(acc)
        for j in range(NJ):
          v = buf[l, pl.ds(16 * j, 16)]
          lo = plsc.bitcast(lax.shift_left(v, 16), jnp.float32)
          hi = plsc.bitcast(
              lax.bitwise_and(v, jnp.int32(-65536)), jnp.float32)
          acc[2 * j] = acc[2 * j] + lo
          acc[2 * j + 1] = acc[2 * j + 1] + hi
        return tuple(acc)

      acc = lax.fori_loop(0, COUNTS[k], accum_body, tuple(acc))

    for c in range(2 * NJ):
      obuf_v[b, pl.ds(c * 16, 16)] = (
          acc[c] * inv_l + bias_v[pl.ds(c * 16, 16)])
    return carry

  lax.fori_loop(0, ROWS_W, row_body, jnp.int32(0))

  @pl.when(first)
  def _():
    pltpu.sync_copy(obuf_v, out1_hbm.at[pl.ds(lw * ROWS_W, ROWS_W)])

  @pl.when(jnp.logical_not(first))
  def _():
    pltpu.sync_copy(obuf_v, out2_hbm.at[pl.ds(lw * ROWS_W, ROWS_W)])


def _mm_kernel(x_ref, wt_ref, o_ref):
  r = jnp.dot(x_ref[...], wt_ref[...], preferred_element_type=jnp.float32)
  def bf16_bits(x):
    return lax.bitcast_convert_type(
        x.astype(jnp.bfloat16).astype(jnp.float32), jnp.int32)
  lo = lax.shift_right_logical(bf16_bits(r[:, :OUT // 2]), 16)
  hi = lax.bitwise_and(bf16_bits(r[:, OUT // 2:]), jnp.int32(-65536))
  o_ref[...] = lax.bitwise_or(hi, lo)


@jax.jit
def _fused(inputs_1, inputs_2, table, W, b):
  wt = W.T[:, jnp.asarray(_PERM)]

  tablew = pl.pallas_call(
      _mm_kernel,
      grid=(V // MBLK,),
      in_specs=[
          pl.BlockSpec((MBLK, D), lambda i: (i, 0)),
          pl.BlockSpec((D, OUT), lambda i: (0, 0)),
      ],
      out_specs=pl.BlockSpec((MBLK, OUT // 2), lambda i: (i, 0)),
      out_shape=jax.ShapeDtypeStruct((V, OUT // 2), jnp.int32),
  )(table, wt)

  mesh = plsc.VectorSubcoreMesh(core_axis_name="c", subcore_axis_name="s",
                                num_cores=NC, num_subcores=NS)
  return pl.kernel(
      _pool_kernel,
      out_type=(jax.ShapeDtypeStruct((B, OUT), jnp.float32),
                jax.ShapeDtypeStruct((B, OUT), jnp.float32)),
      mesh=mesh,
      compiler_params=pltpu.CompilerParams(use_tc_tiling_on_sc=False,
                                           needs_layout_passes=False),
      scratch_types=[
          pltpu.VMEM((ROWS_W * L,), jnp.int32),
          pltpu.VMEM((COUNTS[0], OUT // 2), jnp.int32),
          pltpu.VMEM((COUNTS[1], OUT // 2), jnp.int32),
          pltpu.VMEM((OUT,), jnp.float32),
          pltpu.VMEM((ROWS_W, OUT), jnp.float32),
          pltpu.SemaphoreType.DMA,
          pltpu.SemaphoreType.DMA,
      ],
  )(inputs_1.reshape(-1), inputs_2.reshape(-1), tablew, b)


def kernel(inputs_1, inputs_2, table, W, b):
  return _fused(inputs_1, inputs_2, table, W, b)


# speedup vs baseline: 2.8356x; 1.0023x over previous
import functools

import numpy as np
import jax
import jax.numpy as jnp
from jax import lax
from jax.experimental import pallas as pl
from jax.experimental.pallas import tpu as pltpu
from jax.experimental.pallas import tpu_sc as plsc

V, D, B, L, OUT = 100000, 300, 1024, 200, 256
NC, NS = 2, 16
NW = NC * NS
R = 2 * B
ROWS_W = B // (NW // 2)
G = 104
COUNTS = (G, L - G)
NJ = OUT // 32
MBLK = 5000

_PERM = np.empty((OUT,), np.int32)
for _p in range(128):
  _PERM[_p] = 32 * (_p // 16) + (_p % 16)
  _PERM[128 + _p] = 32 * (_p // 16) + 16 + (_p % 16)


def _pool_kernel(idx1_hbm, idx2_hbm, tw_hbm, bias_hbm, out1_hbm, out2_hbm,
                 idx_v, buf0, buf1, bias_v, obuf_v, sem0, sem1):
  wid = lax.axis_index("s") * NC + lax.axis_index("c")
  first = wid < (NW // 2)
  lw = lax.select(first, wid, wid - NW // 2)
  idx_base = lw * (ROWS_W * L)

  @pl.when(first)
  def _():
    pltpu.sync_copy(idx1_hbm.at[pl.ds(idx_base, ROWS_W * L)], idx_v)

  @pl.when(jnp.logical_not(first))
  def _():
    pltpu.sync_copy(idx2_hbm.at[pl.ds(idx_base, ROWS_W * L)], idx_v)

  pltpu.sync_copy(bias_hbm, bias_v)

  bufs = (buf0, buf1)
  sems = (sem0, sem1)

  def start_gather(b, k):
    pltpu.make_async_copy(
        tw_hbm.at[idx_v.at[pl.ds(b * L + k * G, COUNTS[k])]],
        bufs[k], sems[k]).start()

  start_gather(jnp.int32(0), 0)

  inv_l = jnp.float32(1.0 / L)

  def row_body(b, carry):
    acc = [jnp.zeros((16,), jnp.float32)] * (2 * NJ)
    for k in range(2):
      pltpu.make_async_copy(
          tw_hbm.at[idx_v.at[pl.ds(b * L + k * G, COUNTS[k])]],
          bufs[k], sems[k]).wait()

      if k == 0:
        start_gather(b, 1)
      else:
        @pl.when(b + 1 < ROWS_W)
        def _():
          start_gather(b + 1, 0)

      buf = bufs[k]

      def accum_body(l, acc):
        acc = list(acc)
        for j in range(NJ):
          v = buf[l, pl.ds(16 * j, 16)]
          lo = plsc.bitcast(lax.shift_left(v, 16), jnp.float32)
          hi = plsc.bitcast(
              lax.bitwise_and(v, jnp.int32(-65536)), jnp.float32)
          acc[2 * j] = acc[2 * j] + lo
          acc[2 * j + 1] = acc[2 * j + 1] + hi
        return tuple(acc)

      acc = lax.fori_loop(0, COUNTS[k], accum_body, tuple(acc))

    for c in range(2 * NJ):
      obuf_v[b, pl.ds(c * 16, 16)] = (
          acc[c] * inv_l + bias_v[pl.ds(c * 16, 16)])
    return carry

  lax.fori_loop(0, ROWS_W, row_body, jnp.int32(0))

  @pl.when(first)
  def _():
    pltpu.sync_copy(obuf_v, out1_hbm.at[pl.ds(lw * ROWS_W, ROWS_W)])

  @pl.when(jnp.logical_not(first))
  def _():
    pltpu.sync_copy(obuf_v, out2_hbm.at[pl.ds(lw * ROWS_W, ROWS_W)])


def _mm_kernel(x_ref, wt_ref, o_ref):
  r = jnp.dot(x_ref[...], wt_ref[...], preferred_element_type=jnp.float32)
  def bf16_bits(x):
    return lax.bitcast_convert_type(
        x.astype(jnp.bfloat16).astype(jnp.float32), jnp.int32)
  lo = lax.shift_right_logical(bf16_bits(r[:, :OUT // 2]), 16)
  hi = lax.bitwise_and(bf16_bits(r[:, OUT // 2:]), jnp.int32(-65536))
  o_ref[...] = lax.bitwise_or(hi, lo)


@jax.jit
def _fused(inputs_1, inputs_2, table, W, b):
  wt = W.T[:, jnp.asarray(_PERM)]

  tablew = pl.pallas_call(
      _mm_kernel,
      grid=(V // MBLK,),
      in_specs=[
          pl.BlockSpec((MBLK, D), lambda i: (i, 0)),
          pl.BlockSpec((D, OUT), lambda i: (0, 0)),
      ],
      out_specs=pl.BlockSpec((MBLK, OUT // 2), lambda i: (i, 0)),
      out_shape=jax.ShapeDtypeStruct((V, OUT // 2), jnp.int32),
  )(table, wt)

  mesh = plsc.VectorSubcoreMesh(core_axis_name="c", subcore_axis_name="s",
                                num_cores=NC, num_subcores=NS)
  return pl.kernel(
      _pool_kernel,
      out_type=(jax.ShapeDtypeStruct((B, OUT), jnp.float32),
                jax.ShapeDtypeStruct((B, OUT), jnp.float32)),
      mesh=mesh,
      compiler_params=pltpu.CompilerParams(use_tc_tiling_on_sc=False,
                                           needs_layout_passes=False),
      scratch_types=[
          pltpu.VMEM((ROWS_W * L,), jnp.int32),
          pltpu.VMEM((COUNTS[0], OUT // 2), jnp.int32),
          pltpu.VMEM((COUNTS[1], OUT // 2), jnp.int32),
          pltpu.VMEM((OUT,), jnp.float32),
          pltpu.VMEM((ROWS_W, OUT), jnp.float32),
          pltpu.SemaphoreType.DMA,
          pltpu.SemaphoreType.DMA,
      ],
  )(inputs_1.reshape(-1), inputs_2.reshape(-1), tablew, b)


def kernel(inputs_1, inputs_2, table, W, b):
  return _fused(inputs_1, inputs_2, table, W, b)
